# baseline (speedup 1.0000x reference)
import functools

import jax
import jax.numpy as jnp
from jax import lax
from jax.experimental import pallas as pl
from jax.experimental.pallas import tpu as pltpu
from jax.experimental.pallas import tpu_sc as plsc

B = 16384
D = 32
L = 16
NC = 2
NS = 16
NW = NC * NS
BPW = B // NW
CHUNK = 128
NCH = BPW // CHUNK

_mesh = plsc.VectorSubcoreMesh(core_axis_name="c", subcore_axis_name="s")


@functools.partial(
    pl.kernel,
    mesh=_mesh,
    compiler_params=pltpu.CompilerParams(
        needs_layout_passes=False, use_tc_tiling_on_sc=False),
    out_type=jax.ShapeDtypeStruct((B,), jnp.float32),
    scratch_types=[
        pltpu.VMEM((NCH, CHUNK), jnp.int32),
        pltpu.VMEM((NCH, CHUNK), jnp.int32),
        pltpu.VMEM((NCH, CHUNK, D), jnp.float32),
        pltpu.VMEM((NCH, CHUNK, D), jnp.float32),
        pltpu.VMEM((BPW,), jnp.float32),
        pltpu.SemaphoreType.DMA,
        pltpu.SemaphoreType.DMA,
    ],
)
def _mf_kernel(user_hbm, item_hbm, utab_hbm, itab_hbm, out_hbm,
               uidx, iidx, urows, irows, acc_out, usem, isem):
  wid = lax.axis_index("s") * NC + lax.axis_index("c")
  base = wid * BPW

  for c in range(NCH):
    pltpu.sync_copy(user_hbm.at[pl.ds(base + c * CHUNK, CHUNK)], uidx.at[c])
    pltpu.sync_copy(item_hbm.at[pl.ds(base + c * CHUNK, CHUNK)], iidx.at[c])

  copies = []
  for c in range(NCH):
    copies.append(pltpu.async_copy(utab_hbm.at[uidx.at[c]], urows.at[c], usem))
    copies.append(pltpu.async_copy(itab_hbm.at[iidx.at[c]], irows.at[c], isem))
  for cp in copies:
    cp.wait()

  iota = lax.broadcasted_iota(jnp.int32, (L,), 0)

  def chunk_body(ch, carry):
    c = ch // (CHUNK // L)
    r0 = (ch % (CHUNK // L)) * L
    rows = r0 + iota
    acc = jnp.zeros((L,), jnp.float32)
    for j in range(D):
      col = jnp.full((L,), j, jnp.int32)
      uu = plsc.load_gather(urows.at[c], [rows, col])
      vv = plsc.load_gather(irows.at[c], [rows, col])
      acc = acc + uu * vv
    acc_out[pl.ds(ch * L, L)] = acc
    return carry

  lax.fori_loop(0, BPW // L, chunk_body, 0)

  pltpu.sync_copy(acc_out, out_hbm.at[pl.ds(base, BPW)])


def kernel(user, item, user_table, item_table):
  return _mf_kernel(user.astype(jnp.int32), item.astype(jnp.int32),
                    user_table, item_table)


# speedup vs baseline: 3.8949x; 3.8949x over previous
import functools

import jax
import jax.numpy as jnp
from jax import lax
from jax.experimental import pallas as pl
from jax.experimental.pallas import tpu as pltpu
from jax.experimental.pallas import tpu_sc as plsc

B = 16384
D = 32
L = 16
LANES = 128
NC = 2
NS = 16
NW = NC * NS
BPW = B // NW
EPG = 4
G = BPW // EPG

_mesh = plsc.VectorSubcoreMesh(core_axis_name="c", subcore_axis_name="s")


@functools.partial(
    pl.kernel,
    mesh=_mesh,
    compiler_params=pltpu.CompilerParams(
        needs_layout_passes=False, use_tc_tiling_on_sc=True),
    out_type=jax.ShapeDtypeStruct((B,), jnp.float32),
    scratch_types=[
        pltpu.VMEM((BPW + L,), jnp.int32),
        pltpu.VMEM((BPW + L,), jnp.int32),
        pltpu.VMEM((2, EPG, D, LANES), jnp.float32),
        pltpu.VMEM((2, EPG, D, LANES), jnp.float32),
        pltpu.VMEM((BPW,), jnp.float32),
        pltpu.SemaphoreType.DMA,
        pltpu.SemaphoreType.DMA,
        pltpu.SemaphoreType.DMA,
        pltpu.SemaphoreType.DMA,
    ],
)
def _mf_kernel(user_hbm, item_hbm, utab_hbm, itab_hbm, out_hbm,
               uidx, iidx, ublk, iblk, acc_out,
               usem0, isem0, usem1, isem1):
  wid = lax.axis_index("s") * NC + lax.axis_index("c")
  base = wid * BPW

  pltpu.sync_copy(user_hbm.at[pl.ds(base, BPW)], uidx.at[pl.ds(0, BPW)])
  pltpu.sync_copy(item_hbm.at[pl.ds(base, BPW)], iidx.at[pl.ds(0, BPW)])

  iota = lax.broadcasted_iota(jnp.int32, (L,), 0)
  usems = (usem0, usem1)
  isems = (isem0, isem1)

  def fire_group(g, buf):
    sl = pl.ds(g * EPG, L)
    uvec = (uidx[sl] // LANES) * LANES
    ivec = (iidx[sl] // LANES) * LANES
    for e in range(EPG):
      uk = pl.multiple_of(uvec[e], LANES)
      ik = pl.multiple_of(ivec[e], LANES)
      pltpu.async_copy(
          utab_hbm.at[pl.ds(0, D), pl.ds(uk, LANES)], ublk.at[buf, e],
          usems[buf])
      pltpu.async_copy(
          itab_hbm.at[pl.ds(0, D), pl.ds(ik, LANES)], iblk.at[buf, e],
          isems[buf])

  def drain_group(buf):
    for e in range(EPG):
      pltpu.make_async_copy(
          utab_hbm.at[pl.ds(0, D), pl.ds(0, LANES)], ublk.at[buf, e],
          usems[buf]).wait()
      pltpu.make_async_copy(
          itab_hbm.at[pl.ds(0, D), pl.ds(0, LANES)], iblk.at[buf, e],
          isems[buf]).wait()

  def compute_group(g, buf, lane_base, acc):
    sl = pl.ds(g * EPG, L)
    ul = lax.rem(uidx[sl], jnp.int32(LANES))
    il = lax.rem(iidx[sl], jnp.int32(LANES))
    jlo = iota
    jhi = iota + L
    for e in range(EPG):
      se = jnp.full((L,), e, jnp.int32)
      ue = jnp.full((L,), 1, jnp.int32) * ul[e]
      ie = jnp.full((L,), 1, jnp.int32) * il[e]
      uu = (plsc.load_gather(ublk.at[buf], [se, jlo, ue])
            * plsc.load_gather(iblk.at[buf], [se, jlo, ie])
            + plsc.load_gather(ublk.at[buf], [se, jhi, ue])
            * plsc.load_gather(iblk.at[buf], [se, jhi, ie]))
      acc = jnp.where(iota == lane_base + e, jnp.sum(uu), acc)
    return acc

  fire_group(0, 0)

  def loop_body(t, carry):
    acc = jnp.zeros((L,), jnp.float32)
    for q in range(4):
      g = 4 * t + q
      buf = q % 2
      nbuf = 1 - buf

      @pl.when(g + 1 < G)
      def _():
        fire_group(g + 1, nbuf)

      drain_group(buf)
      acc = compute_group(g, buf, EPG * q, acc)
    acc_out[pl.ds(t * L, L)] = acc
    return carry

  lax.fori_loop(0, G // 4, loop_body, 0)

  pltpu.sync_copy(acc_out, out_hbm.at[pl.ds(base, BPW)])


def kernel(user, item, user_table, item_table):
  return _mf_kernel(user.astype(jnp.int32), item.astype(jnp.int32),
                    user_table.T, item_table.T)


# speedup vs baseline: 4.2609x; 1.0940x over previous
import functools

import jax
import jax.numpy as jnp
from jax import lax
from jax.experimental import pallas as pl
from jax.experimental.pallas import tpu as pltpu
from jax.experimental.pallas import tpu_sc as plsc

B = 16384
D = 32
L = 16
LANES = 128
NC = 2
NS = 16
NW = NC * NS
BPW = B // NW
EPG = 2
NBUF = 4
G = BPW // EPG

_mesh = plsc.VectorSubcoreMesh(core_axis_name="c", subcore_axis_name="s")


@functools.partial(
    pl.kernel,
    mesh=_mesh,
    compiler_params=pltpu.CompilerParams(
        needs_layout_passes=False, use_tc_tiling_on_sc=True),
    out_type=jax.ShapeDtypeStruct((B,), jnp.float32),
    scratch_types=[
        pltpu.VMEM((BPW + L,), jnp.int32),
        pltpu.VMEM((BPW + L,), jnp.int32),
        pltpu.VMEM((NBUF, EPG, D, LANES), jnp.float32),
        pltpu.VMEM((NBUF, EPG, D, LANES), jnp.float32),
        pltpu.VMEM((BPW,), jnp.float32),
        pltpu.SemaphoreType.DMA,
        pltpu.SemaphoreType.DMA,
        pltpu.SemaphoreType.DMA,
        pltpu.SemaphoreType.DMA,
        pltpu.SemaphoreType.DMA,
        pltpu.SemaphoreType.DMA,
        pltpu.SemaphoreType.DMA,
        pltpu.SemaphoreType.DMA,
    ],
)
def _mf_kernel(user_hbm, item_hbm, utab_hbm, itab_hbm, out_hbm,
               uidx, iidx, ublk, iblk, acc_out,
               usem0, isem0, usem1, isem1,
               usem2, isem2, usem3, isem3):
  wid = lax.axis_index("s") * NC + lax.axis_index("c")
  base = wid * BPW

  pltpu.sync_copy(user_hbm.at[pl.ds(base, BPW)], uidx.at[pl.ds(0, BPW)])
  pltpu.sync_copy(item_hbm.at[pl.ds(base, BPW)], iidx.at[pl.ds(0, BPW)])

  iota = lax.broadcasted_iota(jnp.int32, (L,), 0)
  usems = (usem0, usem1, usem2, usem3)
  isems = (isem0, isem1, isem2, isem3)

  def fire_group(g, buf):
    sl = pl.ds(g * EPG, L)
    uvec = (uidx[sl] // LANES) * LANES
    ivec = (iidx[sl] // LANES) * LANES
    for e in range(EPG):
      uk = pl.multiple_of(uvec[e], LANES)
      ik = pl.multiple_of(ivec[e], LANES)
      pltpu.async_copy(
          utab_hbm.at[pl.ds(0, D), pl.ds(uk, LANES)], ublk.at[buf, e],
          usems[buf])
      pltpu.async_copy(
          itab_hbm.at[pl.ds(0, D), pl.ds(ik, LANES)], iblk.at[buf, e],
          isems[buf])

  def drain_group(buf):
    for e in range(EPG):
      pltpu.make_async_copy(
          utab_hbm.at[pl.ds(0, D), pl.ds(0, LANES)], ublk.at[buf, e],
          usems[buf]).wait()
      pltpu.make_async_copy(
          itab_hbm.at[pl.ds(0, D), pl.ds(0, LANES)], iblk.at[buf, e],
          isems[buf]).wait()

  def compute_group(g, buf, lane_base, acc):
    sl = pl.ds(g * EPG, L)
    ul = lax.rem(uidx[sl], jnp.int32(LANES))
    il = lax.rem(iidx[sl], jnp.int32(LANES))
    jlo = iota
    jhi = iota + L
    for e in range(EPG):
      se = jnp.full((L,), e, jnp.int32)
      ue = jnp.full((L,), 1, jnp.int32) * ul[e]
      ie = jnp.full((L,), 1, jnp.int32) * il[e]
      uu = (plsc.load_gather(ublk.at[buf], [se, jlo, ue])
            * plsc.load_gather(iblk.at[buf], [se, jlo, ie])
            + plsc.load_gather(ublk.at[buf], [se, jhi, ue])
            * plsc.load_gather(iblk.at[buf], [se, jhi, ie]))
      acc = jnp.where(iota == lane_base + e, jnp.sum(uu), acc)
    return acc

  for p in range(NBUF - 1):
    fire_group(p, p)

  def loop_body(t, carry):
    acc = jnp.zeros((L,), jnp.float32)
    for q in range(8):
      g = 8 * t + q
      buf = q % NBUF
      fbuf = (q + NBUF - 1) % NBUF

      @pl.when(g + NBUF - 1 < G)
      def _():
        fire_group(g + NBUF - 1, fbuf)

      drain_group(buf)
      acc = compute_group(g, buf, EPG * q, acc)
    acc_out[pl.ds(t * L, L)] = acc
    return carry

  lax.fori_loop(0, G // 8, loop_body, 0)

  pltpu.sync_copy(acc_out, out_hbm.at[pl.ds(base, BPW)])


def kernel(user, item, user_table, item_table):
  return _mf_kernel(user.astype(jnp.int32), item.astype(jnp.int32),
                    user_table.T, item_table.T)
